# baseline (speedup 1.0000x reference)
import functools

import jax
import jax.numpy as jnp
from jax import lax
from jax.experimental import pallas as pl
from jax.experimental.pallas import tpu as pltpu
from jax.experimental.pallas import tpu_sc as plsc

B = 16384
D = 64
H = 128
_GATHER_CHUNK = 128


@functools.cache
def _gather_kernel():
  info = plsc.get_sparse_core_info()
  nc, ns = info.num_cores, info.num_subcores
  nw = nc * ns
  b_per_w = B // nw
  n_chunks = b_per_w // _GATHER_CHUNK
  mesh = plsc.VectorSubcoreMesh(core_axis_name="c", subcore_axis_name="s")

  @functools.partial(
      pl.kernel,
      mesh=mesh,
      compiler_params=pltpu.CompilerParams(use_tc_tiling_on_sc=False),
      out_type=(
          jax.ShapeDtypeStruct((B, D), jnp.float32),
          jax.ShapeDtypeStruct((B, D), jnp.float32),
      ),
      scratch_types=[
          pltpu.VMEM((b_per_w,), jnp.int32),
          pltpu.VMEM((b_per_w,), jnp.int32),
          pltpu.VMEM((b_per_w, D), jnp.float32),
          pltpu.VMEM((b_per_w, D), jnp.float32),
          pltpu.SemaphoreType.DMA,
      ],
  )
  def gather(ut_hbm, it_hbm, uid_hbm, iid_hbm, u_out, i_out,
             uidx_v, iidx_v, urows_v, irows_v, sem):
    wid = lax.axis_index("s") * nc + lax.axis_index("c")
    base = wid * b_per_w
    pltpu.sync_copy(uid_hbm.at[pl.ds(base, b_per_w)], uidx_v)
    pltpu.sync_copy(iid_hbm.at[pl.ds(base, b_per_w)], iidx_v)
    copies = []
    for c in range(n_chunks):
      s = pl.ds(c * _GATHER_CHUNK, _GATHER_CHUNK)
      copies.append(pltpu.async_copy(ut_hbm.at[uidx_v.at[s]], urows_v.at[s], sem))
      copies.append(pltpu.async_copy(it_hbm.at[iidx_v.at[s]], irows_v.at[s], sem))
    for cp in copies:
      cp.wait()
    pltpu.sync_copy(urows_v, u_out.at[pl.ds(base, b_per_w)])
    pltpu.sync_copy(irows_v, i_out.at[pl.ds(base, b_per_w)])

  return gather


def _mlp_body(u_ref, v_ref, w1u_ref, w1i_ref, b1_ref, w2_ref, b2_ref,
              w3_ref, b3_ref, o_ref):
  x = jnp.dot(u_ref[...], w1u_ref[...], preferred_element_type=jnp.float32)
  x += jnp.dot(v_ref[...], w1i_ref[...], preferred_element_type=jnp.float32)
  x = jnp.maximum(x + b1_ref[...], 0.0)
  x = jnp.dot(x, w2_ref[...], preferred_element_type=jnp.float32)
  x = jnp.maximum(x + b2_ref[...], 0.0)
  o_ref[...] = jnp.sum(x * w3_ref[...], axis=1, keepdims=True) + b3_ref[...]


def _mlp(u_vec, i_vec, W1, b1, W2, b2, W3, b3):
  blk = 2048
  grid = (B // blk,)
  full = lambda shape: pl.BlockSpec(shape, lambda i: (0, 0))
  return pl.pallas_call(
      _mlp_body,
      grid=grid,
      in_specs=[
          pl.BlockSpec((blk, D), lambda i: (i, 0)),
          pl.BlockSpec((blk, D), lambda i: (i, 0)),
          full((D, H)),
          full((D, H)),
          full((1, H)),
          full((H, D)),
          full((1, D)),
          full((1, D)),
          full((1, 1)),
      ],
      out_specs=pl.BlockSpec((blk, 1), lambda i: (i, 0)),
      out_shape=jax.ShapeDtypeStruct((B, 1), jnp.float32),
  )(u_vec, i_vec, W1[:D], W1[D:], b1.reshape(1, H), W2, b2.reshape(1, D),
    W3.reshape(1, D), b3.reshape(1, 1))


def kernel(user_id, item_id, user_table, item_table, W1, b1, W2, b2, W3, b3):
  uid = user_id.astype(jnp.int32)
  iid = item_id.astype(jnp.int32)
  u_vec, i_vec = _gather_kernel()(user_table, item_table, uid, iid)
  out = _mlp(u_vec, i_vec, W1, b1, W2, b2, W3, b3)
  return out.reshape(B)
